# baseline (speedup 1.0000x reference)
import jax
import jax.numpy as jnp
import numpy as np
from jax.experimental import pallas as pl
from jax.experimental.pallas import tpu as pltpu

N = 1024
F = 200
B = 8
T = 35
TP = 40
O = 40
KB = 8

_BN_RSQRT = 1.0 / np.sqrt(1.0 + 1e-5)


def _elu(v):
    return jnp.where(v > 0, v, jnp.exp(v) - 1.0)


def _gat_x1t0(x0t, wg_ref, as_ref, ad_ref):
    ht = jax.lax.dot_general(wg_ref[...], x0t, (((1,), (0,)), ((), ())),
                             preferred_element_type=jnp.float32)
    es_row = jax.lax.dot_general(as_ref[...], ht, (((1,), (0,)), ((), ())),
                                 preferred_element_type=jnp.float32)
    ed_row = jax.lax.dot_general(ad_ref[...], ht, (((1,), (0,)), ((), ())),
                                 preferred_element_type=jnp.float32)
    es_col = jnp.transpose(es_row, (1, 0))
    epre = es_col + ed_row
    e = jnp.where(epre > 0, epre, 0.2 * epre)
    ii = jax.lax.broadcasted_iota(jnp.int32, (N, N), 0)
    jj = jax.lax.broadcasted_iota(jnp.int32, (N, N), 1)
    em = jnp.where(ii != jj, e, -1e30)
    m = jnp.max(em, axis=0, keepdims=True)
    p = jnp.exp(em - m)
    s = jnp.sum(p, axis=0, keepdims=True)
    at = p / (s + 1e-16)
    gt = jax.lax.dot_general(ht.astype(jnp.bfloat16), at.astype(jnp.bfloat16),
                             (((1,), (0,)), ((), ())),
                             preferred_element_type=jnp.float32)
    return (x0t + gt).astype(jnp.bfloat16)


def _conv1_kernel(x_ref, x0t_ref, wg_ref, as_ref, ad_ref, w_ref, s_ref,
                  b_ref, o_ref):
    n = pl.program_id(0)
    w16 = w_ref[...].astype(jnp.bfloat16)

    def _finish(y):
        z = _elu(y * s_ref[...] + b_ref[...]).astype(jnp.bfloat16)
        for i in range(O):
            o_ref[0, :, i * N:(i + 1) * N] = z[i * TP:(i + 1) * TP, :]

    @pl.when(n == 0)
    def _():
        x1t0 = _gat_x1t0(x0t_ref[...], wg_ref, as_ref, ad_ref)
        _finish(jax.lax.dot_general(
            w16, x1t0, (((1,), (0,)), ((), ())),
            preferred_element_type=jnp.float32))

    @pl.when(n > 0)
    def _():
        _finish(jax.lax.dot_general(
            w16, x_ref[...].astype(jnp.bfloat16),
            (((1,), (1,)), ((), ())),
            preferred_element_type=jnp.float32))


def _conv2_kernel(p_ref, w_ref, s2_ref, b2_ref, pj_ref, pb_ref, o_ref,
                  acc_ref):
    k = pl.program_id(0)
    part = jax.lax.dot_general(p_ref[...], w_ref[...].astype(jnp.bfloat16),
                               (((1,), (1,)), ((), ())),
                               preferred_element_type=jnp.float32)

    @pl.when(k == 0)
    def _():
        acc_ref[...] = part

    @pl.when(k > 0)
    def _():
        acc_ref[...] = acc_ref[...] + part

    @pl.when(k == KB - 1)
    def _():
        c2 = _elu(acc_ref[...] * s2_ref[...] + b2_ref[...])
        c3 = jax.lax.dot_general(c2, pj_ref[...], (((1,), (0,)), ((), ())),
                                 preferred_element_type=jnp.float32)
        o_ref[...] = c3 + pb_ref[...]


def _mlp_kernel(e_ref, w1_ref, b1_ref, w2_ref, b2_ref, g_ref, be_ref, o_ref):
    y = jax.lax.dot_general(e_ref[...].astype(jnp.bfloat16),
                            w1_ref[...].astype(jnp.bfloat16),
                            (((1,), (1,)), ((), ())),
                            preferred_element_type=jnp.float32) + b1_ref[...]
    z = 0.5 * y * (1.0 + jax.lax.erf(y * np.float32(1.0 / np.sqrt(2.0))))
    z2 = jax.lax.dot_general(z.astype(jnp.bfloat16),
                             w2_ref[...].astype(jnp.bfloat16),
                             (((1,), (1,)), ((), ())),
                             preferred_element_type=jnp.float32) + b2_ref[...]
    yo = y + z2
    mu = jnp.mean(yo, axis=1, keepdims=True)
    d = yo - mu
    var = jnp.mean(d * d, axis=1, keepdims=True)
    o_ref[...] = d * jax.lax.rsqrt(var + 1e-5) * g_ref[...] + be_ref[...]


def _full(shape):
    return pl.BlockSpec(shape, lambda *_: tuple(0 for _ in shape))


def kernel(x, W_gat, att_src, att_dst, b_gat, conv1_w, conv1_b, bn1_g, bn1_b,
           conv2_w, conv2_b, bn2_g, bn2_b, proj_w, proj_b, W1, b1, W2, b2,
           ln_g, ln_b, edge_index):
    del edge_index
    f32 = jnp.float32
    bf16 = jnp.bfloat16
    x2 = x.reshape(B * N, F)
    x0t = x.reshape(B, N, F)[0].T

    w26 = conv1_w[:, 0, 0, :]
    w_eff = sum(jnp.pad(w26, ((0, 0), (j, 4 - j))) for j in range(5)) / 5.0
    rel = jnp.arange(F)[None, :] - 5 * jnp.arange(TP)[:, None]
    valid = (rel >= 0) & (rel < 30) & (jnp.arange(TP)[:, None] < T)
    W3g = jnp.where(valid[None], w_eff[:, jnp.clip(rel, 0, 29)], 0.0)
    W3gf = W3g.reshape(O * TP, F)
    scale1_row = jnp.repeat(bn1_g * _BN_RSQRT, TP).reshape(O * TP, 1)
    bias_bgat = W3gf @ b_gat
    bias_row = ((jnp.repeat(conv1_b, TP) + bias_bgat)
                * scale1_row[:, 0] + jnp.repeat(bn1_b, TP)).reshape(O * TP, 1)
    scale2 = (bn2_g * _BN_RSQRT).reshape(1, O)
    bias2 = (conv2_b * scale2[0] + bn2_b).reshape(1, O)
    projT = proj_w[:, :, 0, 0].T
    pb = proj_b.reshape(1, O)

    P7 = pl.pallas_call(
        _conv1_kernel,
        grid=(B,),
        out_shape=jax.ShapeDtypeStruct((B, TP, O * N), bf16),
        in_specs=[
            pl.BlockSpec((N, F), lambda n: (n, 0)),
            pl.BlockSpec((F, N), lambda n: (0, 0)),
            pl.BlockSpec((F, F), lambda n: (0, 0)),
            pl.BlockSpec((1, F), lambda n: (0, 0)),
            pl.BlockSpec((1, F), lambda n: (0, 0)),
            pl.BlockSpec((O * TP, F), lambda n: (0, 0)),
            pl.BlockSpec((O * TP, 1), lambda n: (0, 0)),
            pl.BlockSpec((O * TP, 1), lambda n: (0, 0)),
        ],
        out_specs=pl.BlockSpec((1, TP, O * N), lambda n: (n, 0, 0)),
    )(x2, x0t, W_gat, att_src.reshape(1, F), att_dst.reshape(1, F),
      W3gf, scale1_row, bias_row)

    P7v = P7.reshape(B * TP, O * N)
    W2r = conv2_w.reshape(O, O * N)
    KBLK = O * N // KB
    c3 = pl.pallas_call(
        _conv2_kernel,
        grid=(KB,),
        out_shape=jax.ShapeDtypeStruct((B * TP, O), f32),
        in_specs=[
            pl.BlockSpec((B * TP, KBLK), lambda k: (0, k)),
            pl.BlockSpec((O, KBLK), lambda k: (0, k)),
            pl.BlockSpec((1, O), lambda k: (0, 0)),
            pl.BlockSpec((1, O), lambda k: (0, 0)),
            pl.BlockSpec((O, O), lambda k: (0, 0)),
            pl.BlockSpec((1, O), lambda k: (0, 0)),
        ],
        out_specs=pl.BlockSpec((B * TP, O), lambda k: (0, 0)),
        scratch_shapes=[pltpu.VMEM((B * TP, O), f32)],
    )(P7v, W2r, scale2, bias2, projT, pb)

    emb = c3.reshape(B, TP, O)[:, :T, :].reshape(B, T * O)
    out = pl.pallas_call(
        _mlp_kernel,
        out_shape=jax.ShapeDtypeStruct((B, N), f32),
        in_specs=[_full((B, T * O)), _full((N, T * O)), _full((1, N)),
                  _full((N, N)), _full((1, N)), _full((1, N)), _full((1, N))],
        out_specs=_full((B, N)),
    )(emb, W1, b1.reshape(1, N), W2, b2.reshape(1, N),
      ln_g.reshape(1, N), ln_b.reshape(1, N))
    return out


# speedup vs baseline: 1.3197x; 1.3197x over previous
import jax
import jax.numpy as jnp
import numpy as np
from jax.experimental import pallas as pl
from jax.experimental.pallas import tpu as pltpu

N = 1024
F = 200
B = 8
T = 35
TP = 40
O = 40
KB = 8

_BN_RSQRT = 1.0 / np.sqrt(1.0 + 1e-5)


def _elu(v):
    return jnp.where(v > 0, v, jnp.exp(v) - 1.0)


def _gat_x1t0(x0t, wg_ref, as_ref, ad_ref):
    ht = jax.lax.dot_general(wg_ref[...], x0t, (((1,), (0,)), ((), ())),
                             preferred_element_type=jnp.float32)
    es_row = jax.lax.dot_general(as_ref[...], ht, (((1,), (0,)), ((), ())),
                                 preferred_element_type=jnp.float32)
    ed_row = jax.lax.dot_general(ad_ref[...], ht, (((1,), (0,)), ((), ())),
                                 preferred_element_type=jnp.float32)
    es_col = jnp.transpose(es_row, (1, 0))
    epre = es_col + ed_row
    e = jnp.where(epre > 0, epre, 0.2 * epre)
    ii = jax.lax.broadcasted_iota(jnp.int32, (N, N), 0)
    jj = jax.lax.broadcasted_iota(jnp.int32, (N, N), 1)
    em = jnp.where(ii != jj, e, -1e30)
    m = jnp.max(em, axis=0, keepdims=True)
    p = jnp.exp(em - m)
    s = jnp.sum(p, axis=0, keepdims=True)
    at = p / (s + 1e-16)
    gt = jax.lax.dot_general(ht.astype(jnp.bfloat16), at.astype(jnp.bfloat16),
                             (((1,), (0,)), ((), ())),
                             preferred_element_type=jnp.float32)
    return (x0t + gt).astype(jnp.bfloat16)


def _conv1_kernel(x_ref, x0t_ref, wg_ref, as_ref, ad_ref, w_ref, s_ref,
                  b_ref, o_ref):
    n = pl.program_id(0)
    w16 = w_ref[...].astype(jnp.bfloat16)

    def _finish(y):
        z = _elu(y * s_ref[...] + b_ref[...]).astype(jnp.bfloat16)
        for i in range(O):
            o_ref[0, :, i * N:(i + 1) * N] = z[i * TP:(i + 1) * TP, :]

    @pl.when(n == 0)
    def _():
        x1t0 = _gat_x1t0(x0t_ref[...], wg_ref, as_ref, ad_ref)
        _finish(jax.lax.dot_general(
            w16, x1t0, (((1,), (0,)), ((), ())),
            preferred_element_type=jnp.float32))

    @pl.when(n > 0)
    def _():
        _finish(jax.lax.dot_general(
            w16, x_ref[...].astype(jnp.bfloat16),
            (((1,), (1,)), ((), ())),
            preferred_element_type=jnp.float32))


def _conv2_kernel(p_ref, w_ref, s2_ref, b2_ref, pj_ref, pb_ref, o_ref,
                  acc_ref):
    k = pl.program_id(0)
    part = jax.lax.dot_general(p_ref[...], w_ref[...],
                               (((1,), (0,)), ((), ())),
                               preferred_element_type=jnp.float32)

    @pl.when(k == 0)
    def _():
        acc_ref[...] = part

    @pl.when(k > 0)
    def _():
        acc_ref[...] = acc_ref[...] + part

    @pl.when(k == KB - 1)
    def _():
        c2 = _elu(acc_ref[...] * s2_ref[...] + b2_ref[...])
        c3 = jax.lax.dot_general(c2, pj_ref[...], (((1,), (0,)), ((), ())),
                                 preferred_element_type=jnp.float32)
        o_ref[...] = c3 + pb_ref[...]


def _mlp_kernel(e_ref, w1_ref, b1_ref, w2_ref, b2_ref, g_ref, be_ref, o_ref):
    y = jax.lax.dot_general(e_ref[...].astype(jnp.bfloat16),
                            w1_ref[...].astype(jnp.bfloat16),
                            (((1,), (1,)), ((), ())),
                            preferred_element_type=jnp.float32) + b1_ref[...]
    z = 0.5 * y * (1.0 + jax.lax.erf(y * np.float32(1.0 / np.sqrt(2.0))))
    z2 = jax.lax.dot_general(z.astype(jnp.bfloat16),
                             w2_ref[...].astype(jnp.bfloat16),
                             (((1,), (1,)), ((), ())),
                             preferred_element_type=jnp.float32) + b2_ref[...]
    yo = y + z2
    mu = jnp.mean(yo, axis=1, keepdims=True)
    d = yo - mu
    var = jnp.mean(d * d, axis=1, keepdims=True)
    o_ref[...] = d * jax.lax.rsqrt(var + 1e-5) * g_ref[...] + be_ref[...]


def _full(shape):
    return pl.BlockSpec(shape, lambda *_: tuple(0 for _ in shape))


def kernel(x, W_gat, att_src, att_dst, b_gat, conv1_w, conv1_b, bn1_g, bn1_b,
           conv2_w, conv2_b, bn2_g, bn2_b, proj_w, proj_b, W1, b1, W2, b2,
           ln_g, ln_b, edge_index):
    del edge_index
    f32 = jnp.float32
    bf16 = jnp.bfloat16
    x2 = x.reshape(B * N, F)
    x0t = x.reshape(B, N, F)[0].T

    w26 = conv1_w[:, 0, 0, :]
    w_eff = sum(jnp.pad(w26, ((0, 0), (j, 4 - j))) for j in range(5)) / 5.0
    rel = jnp.arange(F)[None, :] - 5 * jnp.arange(TP)[:, None]
    valid = (rel >= 0) & (rel < 30) & (jnp.arange(TP)[:, None] < T)
    W3g = jnp.where(valid[None], w_eff[:, jnp.clip(rel, 0, 29)], 0.0)
    W3gf = W3g.reshape(O * TP, F)
    scale1_row = jnp.repeat(bn1_g * _BN_RSQRT, TP).reshape(O * TP, 1)
    bias_bgat = W3gf @ b_gat
    bias_row = ((jnp.repeat(conv1_b, TP) + bias_bgat)
                * scale1_row[:, 0] + jnp.repeat(bn1_b, TP)).reshape(O * TP, 1)
    scale2 = (bn2_g * _BN_RSQRT).reshape(1, O)
    bias2 = (conv2_b * scale2[0] + bn2_b).reshape(1, O)
    projT = proj_w[:, :, 0, 0].T
    pb = proj_b.reshape(1, O)

    P7 = pl.pallas_call(
        _conv1_kernel,
        grid=(B,),
        out_shape=jax.ShapeDtypeStruct((B, TP, O * N), bf16),
        in_specs=[
            pl.BlockSpec((N, F), lambda n: (n, 0)),
            pl.BlockSpec((F, N), lambda n: (0, 0)),
            pl.BlockSpec((F, F), lambda n: (0, 0)),
            pl.BlockSpec((1, F), lambda n: (0, 0)),
            pl.BlockSpec((1, F), lambda n: (0, 0)),
            pl.BlockSpec((O * TP, F), lambda n: (0, 0)),
            pl.BlockSpec((O * TP, 1), lambda n: (0, 0)),
            pl.BlockSpec((O * TP, 1), lambda n: (0, 0)),
        ],
        out_specs=pl.BlockSpec((1, TP, O * N), lambda n: (n, 0, 0)),
    )(x2, x0t, W_gat, att_src.reshape(1, F), att_dst.reshape(1, F),
      W3gf, scale1_row, bias_row)

    P7v = P7.reshape(B * TP, O * N)
    W2flat = jnp.transpose(conv2_w.reshape(O, O * N), (1, 0)).astype(bf16)
    KBLK = O * N // KB
    c3 = pl.pallas_call(
        _conv2_kernel,
        grid=(KB,),
        out_shape=jax.ShapeDtypeStruct((B * TP, O), f32),
        in_specs=[
            pl.BlockSpec((B * TP, KBLK), lambda k: (0, k)),
            pl.BlockSpec((KBLK, O), lambda k: (k, 0)),
            pl.BlockSpec((1, O), lambda k: (0, 0)),
            pl.BlockSpec((1, O), lambda k: (0, 0)),
            pl.BlockSpec((O, O), lambda k: (0, 0)),
            pl.BlockSpec((1, O), lambda k: (0, 0)),
        ],
        out_specs=pl.BlockSpec((B * TP, O), lambda k: (0, 0)),
        scratch_shapes=[pltpu.VMEM((B * TP, O), f32)],
    )(P7v, W2flat, scale2, bias2, projT, pb)

    emb = c3.reshape(B, TP, O)[:, :T, :].reshape(B, T * O)
    out = pl.pallas_call(
        _mlp_kernel,
        out_shape=jax.ShapeDtypeStruct((B, N), f32),
        in_specs=[_full((B, T * O)), _full((N, T * O)), _full((1, N)),
                  _full((N, N)), _full((1, N)), _full((1, N)), _full((1, N))],
        out_specs=_full((B, N)),
    )(emb, W1, b1.reshape(1, N), W2, b2.reshape(1, N),
      ln_g.reshape(1, N), ln_b.reshape(1, N))
    return out


# speedup vs baseline: 1.4098x; 1.0682x over previous
import jax
import jax.numpy as jnp
import numpy as np
from jax.experimental import pallas as pl
from jax.experimental.pallas import tpu as pltpu

N = 1024
F = 200
B = 8
T = 35
TP = 40
O = 40
KB = 8

_BN_RSQRT = 1.0 / np.sqrt(1.0 + 1e-5)


def _elu(v):
    return jnp.where(v > 0, v, jnp.exp(v) - 1.0)


def _gat_x1t0(x0t, wg_ref, as_ref, ad_ref):
    ht = jax.lax.dot_general(wg_ref[...], x0t, (((1,), (0,)), ((), ())),
                             preferred_element_type=jnp.float32)
    es_row = jax.lax.dot_general(as_ref[...], ht, (((1,), (0,)), ((), ())),
                                 preferred_element_type=jnp.float32)
    ed_row = jax.lax.dot_general(ad_ref[...], ht, (((1,), (0,)), ((), ())),
                                 preferred_element_type=jnp.float32)
    es_col = jnp.transpose(es_row, (1, 0))
    epre = es_col + ed_row
    e = jnp.where(epre > 0, epre, 0.2 * epre)
    ii = jax.lax.broadcasted_iota(jnp.int32, (N, N), 0)
    jj = jax.lax.broadcasted_iota(jnp.int32, (N, N), 1)
    em = jnp.where(ii != jj, e, -1e30)
    m = jnp.max(em, axis=0, keepdims=True)
    p = jnp.exp(em - m)
    s = jnp.sum(p, axis=0, keepdims=True)
    at = p / (s + 1e-16)
    gt = jax.lax.dot_general(ht.astype(jnp.bfloat16), at.astype(jnp.bfloat16),
                             (((1,), (0,)), ((), ())),
                             preferred_element_type=jnp.float32)
    return (x0t + gt).astype(jnp.bfloat16)


def _bc_kernel(x_ref, x0t_ref, wg_ref, as_ref, ad_ref, w_ref, s_ref,
               b_ref, w2_ref, s2_ref, b2_ref, pj_ref, pb_ref, o_ref,
               p7_ref, acc_ref):
    n = pl.program_id(0)

    def _finish(y):
        z = _elu(y * s_ref[...] + b_ref[...]).astype(jnp.bfloat16)
        for i in range(O):
            p7_ref[pl.ds(n * TP, TP), i * N:(i + 1) * N] = (
                z[i * TP:(i + 1) * TP, :])

    @pl.when(n == 0)
    def _():
        x1t0 = _gat_x1t0(x0t_ref[...], wg_ref, as_ref, ad_ref)
        _finish(jax.lax.dot_general(
            w_ref[...].astype(jnp.bfloat16), x1t0, (((1,), (0,)), ((), ())),
            preferred_element_type=jnp.float32))

    @pl.when((n > 0) & (n < B))
    def _():
        _finish(jax.lax.dot_general(
            w_ref[...].astype(jnp.bfloat16), x_ref[...].astype(jnp.bfloat16),
            (((1,), (1,)), ((), ())),
            preferred_element_type=jnp.float32))

    @pl.when(n >= B)
    def _():
        k = n - B
        pblk = p7_ref[:, pl.ds(k * (O * N // KB), O * N // KB)]
        part = jax.lax.dot_general(pblk, w2_ref[...],
                                   (((1,), (0,)), ((), ())),
                                   preferred_element_type=jnp.float32)

        @pl.when(k == 0)
        def _():
            acc_ref[...] = part

        @pl.when(k > 0)
        def _():
            acc_ref[...] = acc_ref[...] + part

        @pl.when(k == KB - 1)
        def _():
            c2 = _elu(acc_ref[...] * s2_ref[...] + b2_ref[...])
            c3 = jax.lax.dot_general(c2, pj_ref[...],
                                     (((1,), (0,)), ((), ())),
                                     preferred_element_type=jnp.float32)
            o_ref[...] = c3 + pb_ref[...]


def _mlp_kernel(e_ref, w1_ref, b1_ref, w2_ref, b2_ref, g_ref, be_ref, o_ref):
    y = jax.lax.dot_general(e_ref[...].astype(jnp.bfloat16),
                            w1_ref[...].astype(jnp.bfloat16),
                            (((1,), (1,)), ((), ())),
                            preferred_element_type=jnp.float32) + b1_ref[...]
    z = 0.5 * y * (1.0 + jax.lax.erf(y * np.float32(1.0 / np.sqrt(2.0))))
    z2 = jax.lax.dot_general(z.astype(jnp.bfloat16),
                             w2_ref[...].astype(jnp.bfloat16),
                             (((1,), (1,)), ((), ())),
                             preferred_element_type=jnp.float32) + b2_ref[...]
    yo = y + z2
    mu = jnp.mean(yo, axis=1, keepdims=True)
    d = yo - mu
    var = jnp.mean(d * d, axis=1, keepdims=True)
    o_ref[...] = d * jax.lax.rsqrt(var + 1e-5) * g_ref[...] + be_ref[...]


def _full(shape):
    return pl.BlockSpec(shape, lambda *_: tuple(0 for _ in shape))


def kernel(x, W_gat, att_src, att_dst, b_gat, conv1_w, conv1_b, bn1_g, bn1_b,
           conv2_w, conv2_b, bn2_g, bn2_b, proj_w, proj_b, W1, b1, W2, b2,
           ln_g, ln_b, edge_index):
    del edge_index
    f32 = jnp.float32
    bf16 = jnp.bfloat16
    x2 = x.reshape(B * N, F)
    x0t = x.reshape(B, N, F)[0].T

    w26 = conv1_w[:, 0, 0, :]
    w_eff = sum(jnp.pad(w26, ((0, 0), (j, 4 - j))) for j in range(5)) / 5.0
    rel = jnp.arange(F)[None, :] - 5 * jnp.arange(TP)[:, None]
    valid = (rel >= 0) & (rel < 30) & (jnp.arange(TP)[:, None] < T)
    W3g = jnp.where(valid[None], w_eff[:, jnp.clip(rel, 0, 29)], 0.0)
    W3gf = W3g.reshape(O * TP, F)
    scale1_row = jnp.repeat(bn1_g * _BN_RSQRT, TP).reshape(O * TP, 1)
    bias_bgat = W3gf @ b_gat
    bias_row = ((jnp.repeat(conv1_b, TP) + bias_bgat)
                * scale1_row[:, 0] + jnp.repeat(bn1_b, TP)).reshape(O * TP, 1)
    scale2 = (bn2_g * _BN_RSQRT).reshape(1, O)
    bias2 = (conv2_b * scale2[0] + bn2_b).reshape(1, O)
    projT = proj_w[:, :, 0, 0].T
    pb = proj_b.reshape(1, O)

    W2flat = jnp.transpose(conv2_w.reshape(O, O * N), (1, 0)).astype(bf16)
    KBLK = O * N // KB
    c3 = pl.pallas_call(
        _bc_kernel,
        grid=(B + KB,),
        out_shape=jax.ShapeDtypeStruct((B * TP, O), f32),
        in_specs=[
            pl.BlockSpec((N, F), lambda n: (jnp.minimum(n, B - 1), 0)),
            pl.BlockSpec((F, N), lambda n: (0, 0)),
            pl.BlockSpec((F, F), lambda n: (0, 0)),
            pl.BlockSpec((1, F), lambda n: (0, 0)),
            pl.BlockSpec((1, F), lambda n: (0, 0)),
            pl.BlockSpec((O * TP, F), lambda n: (0, 0)),
            pl.BlockSpec((O * TP, 1), lambda n: (0, 0)),
            pl.BlockSpec((O * TP, 1), lambda n: (0, 0)),
            pl.BlockSpec((KBLK, O), lambda n: (jnp.maximum(n - B, 0), 0)),
            pl.BlockSpec((1, O), lambda n: (0, 0)),
            pl.BlockSpec((1, O), lambda n: (0, 0)),
            pl.BlockSpec((O, O), lambda n: (0, 0)),
            pl.BlockSpec((1, O), lambda n: (0, 0)),
        ],
        out_specs=pl.BlockSpec((B * TP, O), lambda n: (0, 0)),
        scratch_shapes=[pltpu.VMEM((B * TP, O * N), bf16),
                        pltpu.VMEM((B * TP, O), f32)],
    )(x2, x0t, W_gat, att_src.reshape(1, F), att_dst.reshape(1, F),
      W3gf, scale1_row, bias_row, W2flat, scale2, bias2, projT, pb)

    emb = c3.reshape(B, TP, O)[:, :T, :].reshape(B, T * O)
    out = pl.pallas_call(
        _mlp_kernel,
        out_shape=jax.ShapeDtypeStruct((B, N), f32),
        in_specs=[_full((B, T * O)), _full((N, T * O)), _full((1, N)),
                  _full((N, N)), _full((1, N)), _full((1, N)), _full((1, N))],
        out_specs=_full((B, N)),
    )(emb, W1, b1.reshape(1, N), W2, b2.reshape(1, N),
      ln_g.reshape(1, N), ln_b.reshape(1, N))
    return out
